# baseline (speedup 1.0000x reference)
import functools

import jax
import jax.numpy as jnp
from jax import lax
from jax.experimental import pallas as pl
from jax.experimental.pallas import tpu as pltpu
from jax.experimental.pallas import tpu_sc as plsc

D = 64
SEQ = 200
BATCH = 4096
B_TOTAL = BATCH * SEQ
NC = 2
NS = 16
NW = NC * NS
B_PER_W = B_TOTAL // NW
CHUNK = 512
NCHUNKS = B_PER_W // CHUNK
NSTREAM = CHUNK // 128

_mesh = plsc.VectorSubcoreMesh(core_axis_name="c", subcore_axis_name="s")


@functools.partial(
    pl.kernel,
    mesh=_mesh,
    compiler_params=pltpu.CompilerParams(use_tc_tiling_on_sc=False),
    out_type=jax.ShapeDtypeStruct((B_TOTAL // 2, 128), jnp.float32),
    scratch_types=[
        pltpu.VMEM((8, 128), jnp.int32),
        pltpu.VMEM((CHUNK, D), jnp.float32),
        pltpu.VMEM((CHUNK // 2, 128), jnp.float32),
        pltpu.VMEM((SEQ, D), jnp.float32),
        pltpu.SemaphoreType.DMA,
    ],
)
def _embed(table_hbm, idx_hbm, pe_hbm, out_hbm, idx_v, rows_v, out_v, pe_v, sem):
    wid = lax.axis_index("s") * NC + lax.axis_index("c")
    base = wid * B_PER_W
    pltpu.sync_copy(pe_hbm, pe_v)

    def chunk_body(ci, carry):
        cbase = pl.multiple_of(base + ci * CHUNK, CHUNK)
        @pl.when(lax.rem(ci, 2) == 0)
        def _():
            pltpu.sync_copy(
                idx_hbm.at[pl.ds(pl.multiple_of(cbase // 128, 8), 8)], idx_v)
        half = lax.rem(ci, 2) * NSTREAM
        copies = [
            pltpu.async_copy(
                table_hbm.at[idx_v.at[half + j]],
                rows_v.at[pl.ds(j * 128, 128)],
                sem,
            )
            for j in range(NSTREAM)
        ]
        for c in copies:
            c.wait()

        def row_body(r, acc):
            s = lax.rem(cbase + r, SEQ)
            orow = lax.div(r, 2)
            ocol = lax.rem(r, 2) * D
            for j in range(D // 16):
                out_v[orow, pl.ds(ocol + j * 16, 16)] = (
                    rows_v[r, pl.ds(j * 16, 16)] + pe_v[s, pl.ds(j * 16, 16)]
                )
            return acc

        lax.fori_loop(0, CHUNK, row_body, 0)
        pltpu.sync_copy(
            out_v, out_hbm.at[pl.ds(pl.multiple_of(cbase // 2, CHUNK // 2), CHUNK // 2)])
        return carry

    lax.fori_loop(0, NCHUNKS, chunk_body, 0)


def kernel(inputs, table, pos_encoding):
    idx = inputs.reshape(B_TOTAL // 128, 128).astype(jnp.int32)
    pe = pos_encoding[:SEQ]
    out = _embed(table, idx, pe)
    return out.reshape(BATCH, SEQ, D)


# speedup vs baseline: 1.1707x; 1.1707x over previous
import functools

import jax
import jax.numpy as jnp
from jax import lax
from jax.experimental import pallas as pl
from jax.experimental.pallas import tpu as pltpu
from jax.experimental.pallas import tpu_sc as plsc

D = 64
SEQ = 200
BATCH = 4096
B_TOTAL = BATCH * SEQ
NC = 2
NS = 16
NW = NC * NS
B_PER_W = B_TOTAL // NW
CHUNK = 256
NCHUNKS = B_PER_W // CHUNK
NSTREAM = CHUNK // 128

_mesh = plsc.VectorSubcoreMesh(core_axis_name="c", subcore_axis_name="s")


@functools.partial(
    pl.kernel,
    mesh=_mesh,
    out_type=jax.ShapeDtypeStruct((B_TOTAL, D), jnp.float32),
    scratch_types=[
        pltpu.VMEM((8, 128), jnp.int32),
        pltpu.VMEM((CHUNK, 2 * D), jnp.float32),
        pltpu.VMEM((CHUNK, D), jnp.float32),
        pltpu.VMEM((SEQ, D), jnp.float32),
        pltpu.SemaphoreType.DMA,
    ],
)
def _embed(tab_hbm, idx_hbm, pe_hbm, out_hbm, idx_v, rows_v, out_v, pe_v, sem):
    wid = lax.axis_index("s") * NC + lax.axis_index("c")
    base = wid * B_PER_W
    pltpu.sync_copy(pe_hbm, pe_v)

    def chunk_body(ci, carry):
        cbase = pl.multiple_of(base + ci * CHUNK, CHUNK)
        @pl.when(lax.rem(ci, 4) == 0)
        def _():
            pltpu.sync_copy(
                idx_hbm.at[pl.ds(pl.multiple_of(cbase // 128, 8), 8)], idx_v)
        part = lax.rem(ci, 4) * NSTREAM
        copies = [
            pltpu.async_copy(
                tab_hbm.at[idx_v.at[part + j]],
                rows_v.at[pl.ds(j * 128, 128)],
                sem,
            )
            for j in range(NSTREAM)
        ]
        for c in copies:
            c.wait()

        def row_body(r, acc):
            s = lax.rem(cbase + r, SEQ)
            for j in range(D // 16):
                sl = pl.ds(j * 16, 16)
                out_v[r, sl] = rows_v[r, sl] + pe_v[s, sl]
            return acc

        lax.fori_loop(0, CHUNK, row_body, 0)
        pltpu.sync_copy(out_v, out_hbm.at[pl.ds(cbase, CHUNK)])
        return carry

    lax.fori_loop(0, NCHUNKS, chunk_body, 0)


def kernel(inputs, table, pos_encoding):
    tab_pad = jnp.pad(table, ((0, 0), (0, D)))
    idx = inputs.reshape(B_TOTAL // 128, 128).astype(jnp.int32)
    pe = pos_encoding[:SEQ]
    out = _embed(tab_pad, idx, pe)
    return out.reshape(BATCH, SEQ, D)


# speedup vs baseline: 1.3442x; 1.1482x over previous
import functools

import jax
import jax.numpy as jnp
from jax import lax
from jax.experimental import pallas as pl
from jax.experimental.pallas import tpu as pltpu
from jax.experimental.pallas import tpu_sc as plsc

D = 64
SEQ = 200
BATCH = 4096
B_TOTAL = BATCH * SEQ
NC = 2
NS = 16
NW = NC * NS
B_PER_W = B_TOTAL // NW
IDXROWS = B_PER_W // 128
CHUNK = 256
NCHUNKS = B_PER_W // CHUNK
NSTREAM = CHUNK // 128

_mesh = plsc.VectorSubcoreMesh(core_axis_name="c", subcore_axis_name="s")


@functools.partial(
    pl.kernel,
    mesh=_mesh,
    out_type=jax.ShapeDtypeStruct((B_TOTAL, D), jnp.float32),
    scratch_types=[
        pltpu.VMEM((16, 128), jnp.int32),
        pltpu.VMEM((CHUNK, 2 * D), jnp.float32),
        pltpu.VMEM((CHUNK, 2 * D), jnp.float32),
        pltpu.VMEM((CHUNK, D), jnp.float32),
        pltpu.VMEM((SEQ, D), jnp.float32),
        pltpu.SemaphoreType.DMA,
        pltpu.SemaphoreType.DMA,
    ],
)
def _embed(tab_hbm, idx_hbm, pe_hbm, out_hbm,
           idx_v, rows_a, rows_b, out_v, pe_v, sem_a, sem_b):
    wid = lax.axis_index("s") * NC + lax.axis_index("c")
    base = wid * B_PER_W
    pltpu.sync_copy(pe_hbm, pe_v)

    bufs = (rows_a, rows_b)
    sems = (sem_a, sem_b)

    def fire(ci, buf, sem):
        slot = lax.rem(lax.div(ci, 4), 2) * 8

        @pl.when(lax.rem(ci, 4) == 0)
        def _():
            row0 = pl.multiple_of((base + ci * CHUNK) // 128, 8)
            pltpu.sync_copy(idx_hbm.at[pl.ds(row0, 8)],
                            idx_v.at[pl.ds(slot, 8)])
        part = slot + lax.rem(ci, 4) * NSTREAM
        for j in range(NSTREAM):
            pltpu.async_copy(
                tab_hbm.at[idx_v.at[part + j]],
                buf.at[pl.ds(j * 128, 128)],
                sem)

    def process(ci, buf, sem, s0):
        cbase = pl.multiple_of(base + ci * CHUNK, CHUNK)
        for j in range(NSTREAM):
            pltpu.make_async_copy(
                tab_hbm.at[idx_v.at[j]],
                buf.at[pl.ds(j * 128, 128)],
                sem).wait()

        def row_body(r, s):
            for j in range(D // 16):
                sl = pl.ds(j * 16, 16)
                out_v[r, sl] = buf[r, sl] + pe_v[s, sl]
            s = s + 1
            return lax.select(s == SEQ, 0, s)

        s_end = lax.fori_loop(0, CHUNK, row_body, s0)
        pltpu.sync_copy(out_v, out_hbm.at[pl.ds(cbase, CHUNK)])
        return s_end

    fire(0, bufs[0], sems[0])

    def pair_body(k, s):
        c0 = k * 2

        @pl.when(c0 + 1 < NCHUNKS)
        def _():
            fire(c0 + 1, bufs[1], sems[1])
        s = process(c0, bufs[0], sems[0], s)

        @pl.when(c0 + 2 < NCHUNKS)
        def _():
            fire(c0 + 2, bufs[0], sems[0])
        s = process(c0 + 1, bufs[1], sems[1], s)
        return s

    lax.fori_loop(0, NCHUNKS // 2, pair_body, lax.rem(base, SEQ))


def kernel(inputs, table, pos_encoding):
    tab_pad = jnp.pad(table, ((0, 0), (0, D)))
    idx = inputs.reshape(B_TOTAL // 128, 128).astype(jnp.int32)
    pe = pos_encoding[:SEQ]
    out = _embed(tab_pad, idx, pe)
    return out.reshape(BATCH, SEQ, D)


# speedup vs baseline: 1.4829x; 1.1032x over previous
import functools

import jax
import jax.numpy as jnp
from jax import lax
from jax.experimental import pallas as pl
from jax.experimental.pallas import tpu as pltpu
from jax.experimental.pallas import tpu_sc as plsc

D = 64
SEQ = 200
BATCH = 4096
B_TOTAL = BATCH * SEQ
NC = 2
NS = 16
NW = NC * NS
B_PER_W = B_TOTAL // NW
CHUNK = 128
NCHUNKS = B_PER_W // CHUNK

_mesh = plsc.VectorSubcoreMesh(core_axis_name="c", subcore_axis_name="s")


@functools.partial(
    pl.kernel,
    mesh=_mesh,
    out_type=jax.ShapeDtypeStruct((B_TOTAL, D), jnp.float32),
    scratch_types=[
        pltpu.VMEM((16, 128), jnp.int32),
        pltpu.VMEM((CHUNK, 2 * D), jnp.float32),
        pltpu.VMEM((CHUNK, 2 * D), jnp.float32),
        pltpu.VMEM((CHUNK, D), jnp.float32),
        pltpu.VMEM((CHUNK, D), jnp.float32),
        pltpu.VMEM((SEQ, D), jnp.float32),
        pltpu.SemaphoreType.DMA,
        pltpu.SemaphoreType.DMA,
        pltpu.SemaphoreType.DMA,
        pltpu.SemaphoreType.DMA,
    ],
)
def _embed(tab_hbm, idx_hbm, pe_hbm, out_hbm,
           idx_v, rows_a, rows_b, out_a, out_b, pe_v,
           sem_a, sem_b, osem_a, osem_b):
    wid = lax.axis_index("s") * NC + lax.axis_index("c")
    base = wid * B_PER_W
    pltpu.sync_copy(pe_hbm, pe_v)

    def fire(ci, buf, sem):
        slot = lax.rem(lax.div(ci, 8), 2) * 8

        @pl.when(lax.rem(ci, 8) == 0)
        def _():
            row0 = pl.multiple_of((base + ci * CHUNK) // 128, 8)
            pltpu.sync_copy(idx_hbm.at[pl.ds(row0, 8)],
                            idx_v.at[pl.ds(slot, 8)])
        pltpu.async_copy(tab_hbm.at[idx_v.at[slot + lax.rem(ci, 8)]],
                         buf, sem)

    def process(ci, buf, sem, out_v, osem, s0, first):
        cbase = pl.multiple_of(base + ci * CHUNK, CHUNK)
        pltpu.make_async_copy(tab_hbm.at[idx_v.at[0]], buf, sem).wait()
        @pl.when(jnp.logical_not(first))
        def _():
            pltpu.make_async_copy(
                out_v, out_hbm.at[pl.ds(cbase, CHUNK)], osem).wait()

        def row_body(r, s):
            for j in range(D // 16):
                sl = pl.ds(j * 16, 16)
                out_v[r, sl] = buf[r, sl] + pe_v[s, sl]
            s = s + 1
            return lax.select(s == SEQ, 0, s)

        s_end = lax.fori_loop(0, CHUNK, row_body, s0)
        pltpu.async_copy(out_v, out_hbm.at[pl.ds(cbase, CHUNK)], osem)
        return s_end

    fire(0, rows_a, sem_a)

    def pair_body(k, s):
        c0 = k * 2
        first = k == 0

        @pl.when(c0 + 1 < NCHUNKS)
        def _():
            fire(c0 + 1, rows_b, sem_b)
        s = process(c0, rows_a, sem_a, out_a, osem_a, s, first)

        @pl.when(c0 + 2 < NCHUNKS)
        def _():
            fire(c0 + 2, rows_a, sem_a)
        s = process(c0 + 1, rows_b, sem_b, out_b, osem_b, s, first)
        return s

    lax.fori_loop(0, NCHUNKS // 2, pair_body, lax.rem(base, SEQ))
    last = pl.multiple_of(base + (NCHUNKS - 2) * CHUNK, CHUNK)
    pltpu.make_async_copy(out_a, out_hbm.at[pl.ds(last, CHUNK)], osem_a).wait()
    pltpu.make_async_copy(out_b, out_hbm.at[pl.ds(last, CHUNK)], osem_b).wait()


def kernel(inputs, table, pos_encoding):
    tab_pad = jnp.pad(table, ((0, 0), (0, D)))
    idx = inputs.reshape(B_TOTAL // 128, 128).astype(jnp.int32)
    pe = pos_encoding[:SEQ]
    out = _embed(tab_pad, idx, pe)
    return out.reshape(BATCH, SEQ, D)
